# baseline (speedup 1.0000x reference)
import dataclasses
import functools

import jax
import jax.numpy as jnp
from jax import lax
from jax.experimental import pallas as pl
from jax.experimental.pallas import tpu as pltpu
from jax.experimental.pallas import tpu_sc as plsc

N_NODES = 10000
N_PAD = 10240
FEAT = 128
NC, NS, LANES = 2, 16, 16
NW = NC * NS
CHUNK = 128


def _bcast_lane(vec, lane):
    idx = jnp.full((LANES, 1), lane, jnp.int32)
    return lax.gather(
        vec, idx,
        dimension_numbers=lax.GatherDimensionNumbers(
            offset_dims=(), collapsed_slice_dims=(0,), start_index_map=(0,)),
        slice_sizes=(1,),
        mode=lax.GatherScatterMode.PROMISE_IN_BOUNDS)


def _sc_compiler_params():
    cp = pltpu.CompilerParams()
    if "needs_layout_passes" in pltpu.CompilerParams.__dataclass_fields__:
        cp = dataclasses.replace(cp, needs_layout_passes=False)
    return cp


def _spmm_sc(features, src_r, dst_r, w_r, n_chunks):
    mesh = plsc.VectorSubcoreMesh(core_axis_name="c", subcore_axis_name="s")
    stripe = N_PAD // NS

    @functools.partial(
        pl.kernel,
        out_type=jax.ShapeDtypeStruct((NC, N_PAD, FEAT), jnp.float32),
        mesh=mesh,
        scratch_types=[
            pltpu.VMEM((n_chunks // 2, CHUNK), jnp.int32),
            pltpu.VMEM((n_chunks // 2, CHUNK), jnp.int32),
            pltpu.VMEM((n_chunks // 2, CHUNK), jnp.float32),
            pltpu.VMEM((CHUNK, FEAT), jnp.float32),
            pltpu.VMEM((CHUNK, FEAT), jnp.float32),
            pltpu.VMEM_SHARED((N_PAD, FEAT), jnp.float32),
            pltpu.SemaphoreType.DMA,
            pltpu.SemaphoreType.DMA,
        ],
        compiler_params=_sc_compiler_params(),
    )
    def k(feat_hbm, src_hbm, dst_hbm, w_hbm, out_hbm,
          src_v, dst_v, w_v, rows_a, rows_b, gx_sh, sem_a, sem_b):
        rows_v = rows_a
        cid = lax.axis_index("c")
        sid = lax.axis_index("s")
        wid = cid * NS + sid

        zero16 = jnp.zeros((LANES,), jnp.float32)

        @pl.loop(0, CHUNK)
        def _(r):
            for s8 in range(FEAT // LANES):
                rows_v[r, pl.ds(s8 * LANES, LANES)] = zero16

        base = pl.multiple_of(sid * stripe, 8)
        for off in range(0, stripe, CHUNK):
            pltpu.sync_copy(rows_v, gx_sh.at[pl.ds(base + off, CHUNK)])
        plsc.subcore_barrier()

        bufs = (rows_a, rows_b)
        sems = (sem_a, sem_b)
        half = n_chunks // 2
        for h in range(2):
            pltpu.sync_copy(src_hbm.at[wid, pl.ds(h * half, half)], src_v)
            pltpu.sync_copy(dst_hbm.at[wid, pl.ds(h * half, half)], dst_v)
            pltpu.sync_copy(w_hbm.at[wid, pl.ds(h * half, half)], w_v)
            pltpu.async_copy(feat_hbm.at[src_v.at[0]], rows_a, sem_a)

            @pl.loop(0, half // 2)
            def _(jj):
                for p in range(2):
                    j = jj * 2 + p
                    buf, sem = bufs[p], sems[p]
                    nbuf, nsem = bufs[1 - p], sems[1 - p]
                    pltpu.make_async_copy(
                        feat_hbm.at[pl.ds(0, CHUNK)], buf, sem).wait()
                    jn = jnp.minimum(j + 1, half - 1)
                    pltpu.async_copy(feat_hbm.at[src_v.at[jn]], nbuf, nsem)

                    @pl.loop(0, CHUNK)
                    def _(e):
                        w16 = plsc.load_gather(
                            w_v, [jnp.full((LANES,), j, jnp.int32),
                                  jnp.full((LANES,), e, jnp.int32)])
                        for s8 in range(FEAT // LANES):
                            sl = pl.ds(s8 * LANES, LANES)
                            buf[e, sl] = buf[e, sl] * w16

                    pltpu.sync_copy(buf, gx_sh.at[dst_v.at[j]], add=True)

            pltpu.make_async_copy(
                feat_hbm.at[pl.ds(0, CHUNK)], rows_a, sem_a).wait()

        plsc.subcore_barrier()

        pltpu.sync_copy(gx_sh.at[pl.ds(base, stripe)],
                        out_hbm.at[cid, pl.ds(base, stripe)])

    return k(features, src_r, dst_r, w_r)


def _dense_tc(features, gx2, W1, b1, W2, b2):
    w1t = W1.T
    w2t = W2.T
    bsum = (b1 + b2).reshape(1, FEAT)
    blk = 1000

    def body(x_ref, g0_ref, g1_ref, w1_ref, w2_ref, b_ref, o_ref):
        g = g0_ref[...] + g1_ref[...]
        x = x_ref[...]
        p = jnp.dot(g + x, w1_ref[...], preferred_element_type=jnp.float32)
        p = p + jnp.dot(g * x, w2_ref[...], preferred_element_type=jnp.float32)
        p = p + b_ref[...]
        o_ref[...] = jnp.where(p >= 0, p, 0.01 * p)

    return pl.pallas_call(
        body,
        grid=(N_NODES // blk,),
        in_specs=[
            pl.BlockSpec((blk, FEAT), lambda i: (i, 0)),
            pl.BlockSpec((blk, FEAT), lambda i: (i, 0)),
            pl.BlockSpec((blk, FEAT), lambda i: (i, 0)),
            pl.BlockSpec((FEAT, FEAT), lambda i: (0, 0)),
            pl.BlockSpec((FEAT, FEAT), lambda i: (0, 0)),
            pl.BlockSpec((1, FEAT), lambda i: (0, 0)),
        ],
        out_specs=pl.BlockSpec((blk, FEAT), lambda i: (i, 0)),
        out_shape=jax.ShapeDtypeStruct((N_NODES, FEAT), jnp.float32),
    )(features, gx2[0], gx2[1], w1t, w2t, bsum)


def kernel(features, edge_index, edge_weight, W1, b1, W2, b2):
    src = edge_index[0].astype(jnp.int32)
    dst = edge_index[1].astype(jnp.int32)
    w = edge_weight.astype(jnp.float32)
    n_edges = src.shape[0]
    n_chunks = -(-n_edges // (NW * CHUNK))
    n_chunks = -(-n_chunks // 4) * 4
    pad = NW * n_chunks * CHUNK - n_edges
    src_r = jnp.pad(src, (0, pad)).reshape(NW, n_chunks, CHUNK)
    dst_r = jnp.pad(dst, (0, pad)).reshape(NW, n_chunks, CHUNK)
    w_r = jnp.pad(w, (0, pad)).reshape(NW, n_chunks, CHUNK)
    gx2 = _spmm_sc(features, src_r, dst_r, w_r, n_chunks)
    return _dense_tc(features, gx2, W1, b1, W2, b2)


# speedup vs baseline: 1.1415x; 1.1415x over previous
import dataclasses
import functools

import jax
import jax.numpy as jnp
from jax import lax
from jax.experimental import pallas as pl
from jax.experimental.pallas import tpu as pltpu
from jax.experimental.pallas import tpu_sc as plsc

N_NODES = 10000
N_PAD = 10240
FEAT = 128
NC, NS, LANES = 2, 16, 16
NW = NC * NS
CHUNK = 128
PHASE = 40
CH0 = 3 * PHASE
CH1 = 1 * PHASE


def _sc_compiler_params():
    cp = pltpu.CompilerParams()
    if "needs_layout_passes" in pltpu.CompilerParams.__dataclass_fields__:
        cp = dataclasses.replace(cp, needs_layout_passes=False)
    return cp


def _spmm_sc(features, src_r, dst_r, w_r):
    mesh = plsc.VectorSubcoreMesh(core_axis_name="c", subcore_axis_name="s")
    stripe = N_PAD // NS

    @functools.partial(
        pl.kernel,
        out_type=jax.ShapeDtypeStruct((NC, N_PAD, FEAT), jnp.float32),
        mesh=mesh,
        scratch_types=[
            pltpu.VMEM((PHASE, CHUNK), jnp.int32),
            pltpu.VMEM((PHASE, CHUNK), jnp.int32),
            pltpu.VMEM((PHASE, CHUNK), jnp.float32),
            pltpu.VMEM((CHUNK, FEAT), jnp.float32),
            pltpu.VMEM((CHUNK, FEAT), jnp.float32),
            pltpu.VMEM_SHARED((N_PAD, FEAT), jnp.float32),
            pltpu.SemaphoreType.DMA,
            pltpu.SemaphoreType.DMA,
        ],
        compiler_params=_sc_compiler_params(),
    )
    def k(feat_hbm, src_hbm, dst_hbm, w_hbm, out_hbm,
          src_v, dst_v, w_v, rows_a, rows_b, gx_sh, sem_a, sem_b):
        cid = lax.axis_index("c")
        sid = lax.axis_index("s")

        zero16 = jnp.zeros((LANES,), jnp.float32)

        @pl.loop(0, CHUNK)
        def _(r):
            for s8 in range(FEAT // LANES):
                rows_a[r, pl.ds(s8 * LANES, LANES)] = zero16

        base = pl.multiple_of(sid * stripe, 8)
        for off in range(0, stripe, CHUNK):
            pltpu.sync_copy(rows_a, gx_sh.at[pl.ds(base + off, CHUNK)])
        plsc.subcore_barrier()

        bufs = (rows_a, rows_b)
        sems = (sem_a, sem_b)

        def pipeline(base_chunk, phases):
            for h in range(phases):
                start = base_chunk + h * PHASE
                pltpu.sync_copy(src_hbm.at[pl.ds(start, PHASE)], src_v)
                pltpu.sync_copy(dst_hbm.at[pl.ds(start, PHASE)], dst_v)
                pltpu.sync_copy(w_hbm.at[pl.ds(start, PHASE)], w_v)
                pltpu.async_copy(feat_hbm.at[src_v.at[0]], rows_a, sem_a)

                @pl.loop(0, PHASE // 2)
                def _(jj):
                    for p in range(2):
                        j = jj * 2 + p
                        buf, sem = bufs[p], sems[p]
                        nbuf, nsem = bufs[1 - p], sems[1 - p]
                        pltpu.make_async_copy(
                            feat_hbm.at[pl.ds(0, CHUNK)], buf, sem).wait()
                        jn = jnp.minimum(j + 1, PHASE - 1)
                        pltpu.async_copy(feat_hbm.at[src_v.at[jn]], nbuf, nsem)

                        @pl.loop(0, CHUNK)
                        def _(e):
                            w16 = plsc.load_gather(
                                w_v, [jnp.full((LANES,), j, jnp.int32),
                                      jnp.full((LANES,), e, jnp.int32)])
                            for s8 in range(FEAT // LANES):
                                sl = pl.ds(s8 * LANES, LANES)
                                buf[e, sl] = buf[e, sl] * w16

                        pltpu.sync_copy(buf, gx_sh.at[dst_v.at[j]], add=True)

                pltpu.make_async_copy(
                    feat_hbm.at[pl.ds(0, CHUNK)], rows_a, sem_a).wait()

        @pl.when(cid == 0)
        def _():
            pipeline(sid * CH0, CH0 // PHASE)

        @pl.when(cid == 1)
        def _():
            pipeline(NS * CH0 + sid * CH1, CH1 // PHASE)

        plsc.subcore_barrier()

        pltpu.sync_copy(gx_sh.at[pl.ds(base, stripe)],
                        out_hbm.at[cid, pl.ds(base, stripe)])

    return k(features, src_r, dst_r, w_r)


def _dense_tc(features, gx2, W1, b1, W2, b2):
    w1t = W1.T
    w2t = W2.T
    bsum = (b1 + b2).reshape(1, FEAT)
    blk = 1000

    def body(x_ref, g0_ref, g1_ref, w1_ref, w2_ref, b_ref, o_ref):
        g = g0_ref[...] + g1_ref[...]
        x = x_ref[...]
        p = jnp.dot(g + x, w1_ref[...], preferred_element_type=jnp.float32)
        p = p + jnp.dot(g * x, w2_ref[...], preferred_element_type=jnp.float32)
        p = p + b_ref[...]
        o_ref[...] = jnp.where(p >= 0, p, 0.01 * p)

    return pl.pallas_call(
        body,
        grid=(N_NODES // blk,),
        in_specs=[
            pl.BlockSpec((blk, FEAT), lambda i: (i, 0)),
            pl.BlockSpec((blk, FEAT), lambda i: (i, 0)),
            pl.BlockSpec((blk, FEAT), lambda i: (i, 0)),
            pl.BlockSpec((FEAT, FEAT), lambda i: (0, 0)),
            pl.BlockSpec((FEAT, FEAT), lambda i: (0, 0)),
            pl.BlockSpec((1, FEAT), lambda i: (0, 0)),
        ],
        out_specs=pl.BlockSpec((blk, FEAT), lambda i: (i, 0)),
        out_shape=jax.ShapeDtypeStruct((N_NODES, FEAT), jnp.float32),
    )(features, gx2[0], gx2[1], w1t, w2t, bsum)


def kernel(features, edge_index, edge_weight, W1, b1, W2, b2):
    src = edge_index[0].astype(jnp.int32)
    dst = edge_index[1].astype(jnp.int32)
    w = edge_weight.astype(jnp.float32)
    n_edges = src.shape[0]
    cap_chunks = NS * (CH0 + CH1)
    assert cap_chunks * CHUNK >= n_edges, "edge capacity too small"
    pad = cap_chunks * CHUNK - n_edges
    src_r = jnp.pad(src, (0, pad)).reshape(cap_chunks, CHUNK)
    dst_r = jnp.pad(dst, (0, pad)).reshape(cap_chunks, CHUNK)
    w_r = jnp.pad(w, (0, pad)).reshape(cap_chunks, CHUNK)
    gx2 = _spmm_sc(features, src_r, dst_r, w_r)
    return _dense_tc(features, gx2, W1, b1, W2, b2)


# speedup vs baseline: 1.1784x; 1.0323x over previous
import dataclasses
import functools

import jax
import jax.numpy as jnp
from jax import lax
from jax.experimental import pallas as pl
from jax.experimental.pallas import tpu as pltpu
from jax.experimental.pallas import tpu_sc as plsc

N_NODES = 10000
N_PAD = 10240
FEAT = 128
NC, NS, LANES = 2, 16, 16
NW = NC * NS
CHUNK = 128
PHASE = 40
CH0 = 3 * PHASE
CH1 = 1 * PHASE
STRIPE = N_PAD // NS


def _sc_compiler_params():
    cp = pltpu.CompilerParams()
    if "needs_layout_passes" in pltpu.CompilerParams.__dataclass_fields__:
        cp = dataclasses.replace(cp, needs_layout_passes=False)
    return cp


def _spmm_sc(features, src_r, dst_r, w_r):
    mesh = plsc.VectorSubcoreMesh(core_axis_name="c", subcore_axis_name="s")

    @functools.partial(
        pl.kernel,
        out_type=jax.ShapeDtypeStruct((NC * N_PAD, FEAT), jnp.float32),
        mesh=mesh,
        scratch_types=[
            pltpu.VMEM((PHASE, CHUNK), jnp.int32),
            pltpu.VMEM((PHASE, CHUNK), jnp.int32),
            pltpu.VMEM((PHASE, CHUNK), jnp.float32),
            pltpu.VMEM((CHUNK, FEAT), jnp.float32),
            pltpu.VMEM((CHUNK, FEAT), jnp.float32),
            pltpu.VMEM_SHARED((N_PAD, FEAT), jnp.float32),
            pltpu.SemaphoreType.DMA,
            pltpu.SemaphoreType.DMA,
        ],
        compiler_params=_sc_compiler_params(),
    )
    def k(feat_hbm, src_hbm, dst_hbm, w_hbm, out_hbm,
          src_v, dst_v, w_v, rows_a, rows_b, gx_sh, sem_a, sem_b):
        cid = lax.axis_index("c")
        sid = lax.axis_index("s")
        base = pl.multiple_of(sid * STRIPE, 8)

        zero16 = jnp.zeros((LANES,), jnp.float32)

        @pl.loop(0, CHUNK)
        def _(r):
            for c in range(FEAT // LANES):
                rows_a[r, pl.ds(c * LANES, LANES)] = zero16

        @pl.loop(0, STRIPE // CHUNK)
        def _(t):
            off = pl.multiple_of(base + t * CHUNK, 8)
            pltpu.sync_copy(rows_a, gx_sh.at[pl.ds(off, CHUNK)])

        plsc.subcore_barrier()

        def scale_and_scatter(buf, j, jj):
            @pl.loop(0, CHUNK)
            def _(r):
                ridx = jnp.zeros((LANES,), jnp.int32) + r
                jidx = jnp.zeros((LANES,), jnp.int32) + jj
                wv = plsc.load_gather(w_v, [jidx, ridx])
                for c in range(FEAT // LANES):
                    sl = pl.ds(c * LANES, LANES)
                    buf[r, sl] = buf[r, sl] * wv

            pltpu.sync_copy(buf, gx_sh.at[dst_v.at[j]], add=True)

        def wait(buf, sem):
            pltpu.make_async_copy(
                feat_hbm.at[pl.ds(0, CHUNK)], buf, sem).wait()

        def pipeline(base_chunk, phases):
            for h in range(phases):
                start = base_chunk + h * PHASE
                pltpu.sync_copy(src_hbm.at[pl.ds(start, PHASE)], src_v)
                pltpu.sync_copy(dst_hbm.at[pl.ds(start, PHASE)], dst_v)
                pltpu.sync_copy(w_hbm.at[pl.ds(start, PHASE)], w_v)

                pltpu.async_copy(feat_hbm.at[src_v.at[0]], rows_a, sem_a)

                @pl.loop(0, PHASE // 2)
                def _(i):
                    j = 2 * i
                    pltpu.async_copy(feat_hbm.at[src_v.at[j + 1]], rows_b,
                                     sem_b)
                    wait(rows_a, sem_a)
                    scale_and_scatter(rows_a, j, j)

                    @pl.when(i < PHASE // 2 - 1)
                    def _():
                        pltpu.async_copy(feat_hbm.at[src_v.at[j + 2]],
                                         rows_a, sem_a)

                    wait(rows_b, sem_b)
                    scale_and_scatter(rows_b, j + 1, j + 1)

        @pl.when(cid == 0)
        def _():
            pipeline(sid * CH0, CH0 // PHASE)

        @pl.when(cid == 1)
        def _():
            pipeline(NS * CH0 + sid * CH1, CH1 // PHASE)

        plsc.subcore_barrier()

        @pl.loop(0, STRIPE // CHUNK)
        def _(t):
            off = pl.multiple_of(base + t * CHUNK, 8)
            hoff = pl.multiple_of(cid * N_PAD + base + t * CHUNK, 8)
            pltpu.sync_copy(gx_sh.at[pl.ds(off, CHUNK)],
                            out_hbm.at[pl.ds(hoff, CHUNK)])

    return k(features, src_r, dst_r, w_r)


def _dense_tc(features, g0, g1, W1, b1, W2, b2):
    w1t = W1.T
    w2t = W2.T
    bsum = (b1 + b2).reshape(1, FEAT)
    blk = 1000

    def body(x_ref, g0_ref, g1_ref, w1_ref, w2_ref, b_ref, o_ref):
        g = g0_ref[...] + g1_ref[...]
        x = x_ref[...]
        p = jnp.dot(g + x, w1_ref[...], preferred_element_type=jnp.float32)
        p = p + jnp.dot(g * x, w2_ref[...], preferred_element_type=jnp.float32)
        p = p + b_ref[...]
        o_ref[...] = jnp.where(p >= 0, p, 0.01 * p)

    return pl.pallas_call(
        body,
        grid=(N_NODES // blk,),
        in_specs=[
            pl.BlockSpec((blk, FEAT), lambda i: (i, 0)),
            pl.BlockSpec((blk, FEAT), lambda i: (i, 0)),
            pl.BlockSpec((blk, FEAT), lambda i: (i, 0)),
            pl.BlockSpec((FEAT, FEAT), lambda i: (0, 0)),
            pl.BlockSpec((FEAT, FEAT), lambda i: (0, 0)),
            pl.BlockSpec((1, FEAT), lambda i: (0, 0)),
        ],
        out_specs=pl.BlockSpec((blk, FEAT), lambda i: (i, 0)),
        out_shape=jax.ShapeDtypeStruct((N_NODES, FEAT), jnp.float32),
    )(features, g0, g1, w1t, w2t, bsum)


def kernel(features, edge_index, edge_weight, W1, b1, W2, b2):
    src = edge_index[0].astype(jnp.int32)
    dst = edge_index[1].astype(jnp.int32)
    w = edge_weight.astype(jnp.float32)
    n_edges = src.shape[0]
    cap_chunks = NS * (CH0 + CH1)
    assert cap_chunks * CHUNK >= n_edges, "edge capacity too small"
    pad = cap_chunks * CHUNK - n_edges
    src_r = jnp.pad(src, (0, pad)).reshape(cap_chunks, CHUNK)
    dst_r = jnp.pad(dst, (0, pad)).reshape(cap_chunks, CHUNK)
    w_r = jnp.pad(w, (0, pad)).reshape(cap_chunks, CHUNK)
    gx_flat = _spmm_sc(features, src_r, dst_r, w_r)
    g0 = gx_flat[:N_NODES]
    g1 = gx_flat[N_PAD:N_PAD + N_NODES]
    return _dense_tc(features, g0, g1, W1, b1, W2, b2)
